# Initial kernel scaffold; baseline (speedup 1.0000x reference)
import jax
import jax.numpy as jnp
from jax.experimental import pallas as pl


def kernel(x, edge_index, edge_attr, W1, b1, W2, b2):
    raise NotImplementedError("write your pallas kernel here")


# speedup vs baseline: 8.6659x; 8.6659x over previous
import functools

import jax
import jax.numpy as jnp
from jax import lax
from jax.experimental import pallas as pl
from jax.experimental.pallas import tpu as pltpu
from jax.experimental.pallas import tpu_sc as plsc

N = 10000
D = 128
NPAD = 10240
NC, NS = 2, 16
NT = NC * NS
CHUNK = 128
CPT = 80
NPHASE = 2
CPH = CPT // NPHASE
EPT = CPT * CHUNK
EPAD = NT * EPT
ACCN = 10112
ROWS_PT = ACCN // NS
_RSZ = (128, 128, 128, 128, 120)

_mesh = plsc.VectorSubcoreMesh(
    core_axis_name="c", subcore_axis_name="s", num_cores=NC, num_subcores=NS)


@functools.partial(
    pl.kernel,
    out_type=jax.ShapeDtypeStruct((NT, NPAD), jnp.float32),
    mesh=_mesh,
    scratch_types=[
        pltpu.VMEM((CPT, CHUNK), jnp.int32),
        pltpu.VMEM((NPAD,), jnp.float32),
    ],
    compiler_params=pltpu.CompilerParams(needs_layout_passes=False),
)
def _deg_kernel(dst2, out, dst_v, deg_v):
    c = lax.axis_index("c")
    s = lax.axis_index("s")
    w = c * NS + s

    def zero(i, carry):
        deg_v[pl.ds(i * 16, 16)] = jnp.zeros((16,), jnp.float32)
        return carry
    lax.fori_loop(0, NPAD // 16, zero, 0)

    pltpu.sync_copy(dst2.at[pl.ds(w * CPT, CPT)], dst_v)
    ones = jnp.ones((16,), jnp.float32)

    def body(j, carry):
        for k in range(CHUNK // 16):
            idx = dst_v[j, pl.ds(k * 16, 16)]
            plsc.addupdate_scatter(deg_v, [idx], ones)
        return carry
    lax.fori_loop(0, CPT, body, 0)

    pltpu.sync_copy(deg_v, out.at[w])


@functools.partial(
    pl.kernel,
    out_type=jax.ShapeDtypeStruct((NC, NPAD, D), jnp.float32),
    mesh=_mesh,
    scratch_types=[
        pltpu.VMEM((CPH, CHUNK), jnp.int32),
        pltpu.VMEM((CPH, CHUNK), jnp.int32),
        pltpu.VMEM((CHUNK, D), jnp.float32),
        pltpu.VMEM((CHUNK, D), jnp.float32),
        pltpu.VMEM_SHARED((ACCN, D), jnp.float32),
        pltpu.SemaphoreType.DMA,
        pltpu.SemaphoreType.DMA,
    ],
)
def _msg_kernel(ht, src2, dst2, out, src_v, dst_v, buf0, buf1, acc, sem0, sem1):
    c = lax.axis_index("c")
    s = lax.axis_index("s")
    w = c * NS + s
    r0 = s * ROWS_PT

    off = 0
    for sz in _RSZ:
        pltpu.sync_copy(ht.at[pl.ds(r0 + off, sz)], buf0.at[pl.ds(0, sz)])
        pltpu.sync_copy(buf0.at[pl.ds(0, sz)], acc.at[pl.ds(r0 + off, sz)])
        off += sz
    plsc.subcore_barrier()

    for phase in range(NPHASE):
        base = w * CPT + phase * CPH
        pltpu.sync_copy(src2.at[pl.ds(base, CPH)], src_v)
        pltpu.sync_copy(dst2.at[pl.ds(base, CPH)], dst_v)

        pltpu.make_async_copy(ht.at[src_v.at[0]], buf0, sem0).start()
        pltpu.make_async_copy(ht.at[src_v.at[1]], buf1, sem1).start()

        def body(j2, carry):
            for b, (buf, sem) in enumerate(((buf0, sem0), (buf1, sem1))):
                j = j2 * 2 + b
                pltpu.make_async_copy(ht.at[src_v.at[j]], buf, sem).wait()
                pltpu.sync_copy(buf, acc.at[dst_v.at[j]], add=True)
                nj = j + 2

                @pl.when(nj < CPH)
                def _():
                    pltpu.make_async_copy(ht.at[src_v.at[nj]], buf, sem).start()
            return carry
        lax.fori_loop(0, CPH // 2, body, 0)

    plsc.subcore_barrier()
    off = 0
    for sz in _RSZ:
        pltpu.sync_copy(acc.at[pl.ds(r0 + off, sz)], buf0.at[pl.ds(0, sz)])
        pltpu.sync_copy(buf0.at[pl.ds(0, sz)], out.at[c, pl.ds(r0 + off, sz)])
        off += sz


_RB = 1024


def _dinv_of(degp_blk):
    return lax.rsqrt(jnp.sum(degp_blk, axis=0) + 1.0)


def _pre_body(degp_ref, x_ref, w_ref, out_ref):
    dinv = _dinv_of(degp_ref[...])
    h = jnp.dot(x_ref[...], w_ref[...], preferred_element_type=jnp.float32,
                precision=lax.Precision.HIGHEST)
    out_ref[...] = h * dinv[:, None]


def _mid_body(degp_ref, p_ref, ht_ref, w_ref, b_ref, out_ref):
    dinv = _dinv_of(degp_ref[...])
    a = (p_ref[0] + p_ref[1] - ht_ref[...]) * dinv[:, None] + b_ref[...]
    g = jnp.maximum(a, 0.0)
    h = jnp.dot(g, w_ref[...], preferred_element_type=jnp.float32,
                precision=lax.Precision.HIGHEST)
    out_ref[...] = h * dinv[:, None]


def _post_body(degp_ref, p_ref, ht_ref, b_ref, out_ref):
    dinv = _dinv_of(degp_ref[...])
    out_ref[...] = ((p_ref[0] + p_ref[1] - ht_ref[...]) * dinv[:, None]
                    + b_ref[...])


_degp_spec = pl.BlockSpec((NT, _RB), lambda i: (0, i))
_row_spec = pl.BlockSpec((_RB, D), lambda i: (i, 0))
_p_spec = pl.BlockSpec((NC, _RB, D), lambda i: (0, i, 0))
_w_spec = pl.BlockSpec((D, D), lambda i: (0, 0))
_b_spec = pl.BlockSpec((1, D), lambda i: (0, 0))
_GRID = (NPAD // _RB,)
_row_out = jax.ShapeDtypeStruct((NPAD, D), jnp.float32)

_pre = pl.pallas_call(
    _pre_body, grid=_GRID,
    in_specs=[_degp_spec, _row_spec, _w_spec],
    out_specs=_row_spec, out_shape=_row_out)

_mid = pl.pallas_call(
    _mid_body, grid=_GRID,
    in_specs=[_degp_spec, _p_spec, _row_spec, _w_spec, _b_spec],
    out_specs=_row_spec, out_shape=_row_out)

_post = pl.pallas_call(
    _post_body, grid=_GRID,
    in_specs=[_degp_spec, _p_spec, _row_spec, _b_spec],
    out_specs=_row_spec, out_shape=_row_out)


def kernel(x, edge_index, edge_attr, W1, b1, W2, b2):
    n, d = x.shape
    src = edge_index[0]
    dst = edge_index[1]
    e = src.shape[0]

    x_p = jnp.zeros((NPAD, d), x.dtype).at[:n].set(x)
    pad = jnp.full((EPAD - e,), n, dtype=src.dtype)
    src2 = jnp.concatenate([src, pad]).reshape(NT * CPT, CHUNK)
    dst2 = jnp.concatenate([dst, pad]).reshape(NT * CPT, CHUNK)
    b1r = b1.reshape(1, d)
    b2r = b2.reshape(1, d)

    degp = _deg_kernel(dst2)
    ht1 = _pre(degp, x_p, W1)
    p1 = _msg_kernel(ht1, src2, dst2)
    ht2 = _mid(degp, p1, ht1, W2, b1r)
    p2 = _msg_kernel(ht2, src2, dst2)
    out = _post(degp, p2, ht2, b2r)
    return out[:n]


# speedup vs baseline: 31.7852x; 3.6678x over previous
import functools

import jax
import jax.numpy as jnp
from jax import lax
from jax.experimental import pallas as pl
from jax.experimental.pallas import tpu as pltpu
from jax.experimental.pallas import tpu_sc as plsc

N = 10000
D = 128
NPAD = 10240
NC, NS = 2, 16
NT = NC * NS
CHUNK = 128
CPT = 80
NPHASE = 2
CPH = CPT // NPHASE
EPT = CPT * CHUNK
EPAD = NT * EPT
ACCN = 10112
ROWS_PT = ACCN // NS
_RSZ = (128, 128, 128, 128, 120)

_mesh = plsc.VectorSubcoreMesh(
    core_axis_name="c", subcore_axis_name="s", num_cores=NC, num_subcores=NS)


@functools.partial(
    pl.kernel,
    out_type=jax.ShapeDtypeStruct((NT, NPAD), jnp.float32),
    mesh=_mesh,
    scratch_types=[
        pltpu.VMEM((CPT, CHUNK), jnp.int32),
        pltpu.VMEM((NPAD,), jnp.float32),
    ],
    compiler_params=pltpu.CompilerParams(needs_layout_passes=False),
)
def _deg_kernel(dst2, out, dst_v, deg_v):
    c = lax.axis_index("c")
    s = lax.axis_index("s")
    w = c * NS + s

    def zero(i, carry):
        deg_v[pl.ds(i * 16, 16)] = jnp.zeros((16,), jnp.float32)
        return carry
    lax.fori_loop(0, NPAD // 16, zero, 0)

    pltpu.sync_copy(dst2.at[pl.ds(w * CPT, CPT)], dst_v)
    ones = jnp.ones((16,), jnp.float32)

    def body(j, carry):
        for k in range(CHUNK // 16):
            idx = dst_v[j, pl.ds(k * 16, 16)]
            plsc.addupdate_scatter(deg_v, [idx], ones)
        return carry
    lax.fori_loop(0, CPT, body, 0)

    pltpu.sync_copy(deg_v, out.at[w])


@functools.partial(
    pl.kernel,
    out_type=jax.ShapeDtypeStruct((NC, NPAD, D), jnp.float32),
    mesh=_mesh,
    scratch_types=[
        pltpu.VMEM((CPH, CHUNK), jnp.int32),
        pltpu.VMEM((CPH, CHUNK), jnp.int32),
        pltpu.VMEM((CHUNK, D), jnp.float32),
        pltpu.VMEM((CHUNK, D), jnp.float32),
        pltpu.VMEM_SHARED((ACCN, D), jnp.float32),
        pltpu.SemaphoreType.DMA,
        pltpu.SemaphoreType.DMA,
    ],
)
def _msg_kernel(ht, src2, dst2, out, src_v, dst_v, buf0, buf1, acc, sem0, sem1):
    c = lax.axis_index("c")
    s = lax.axis_index("s")
    w = c * NS + s
    r0 = s * ROWS_PT

    off = 0
    for sz in _RSZ:
        pltpu.sync_copy(ht.at[pl.ds(r0 + off, sz)], buf0.at[pl.ds(0, sz)])
        pltpu.sync_copy(buf0.at[pl.ds(0, sz)], acc.at[pl.ds(r0 + off, sz)])
        off += sz
    plsc.subcore_barrier()

    for phase in range(NPHASE):
        base = w * CPT + phase * CPH
        pltpu.sync_copy(src2.at[pl.ds(base, CPH)], src_v)
        pltpu.sync_copy(dst2.at[pl.ds(base, CPH)], dst_v)

        pltpu.make_async_copy(ht.at[src_v.at[0]], buf0, sem0).start()
        pltpu.make_async_copy(ht.at[src_v.at[1]], buf1, sem1).start()

        def body(j2, carry):
            for b, (buf, sem) in enumerate(((buf0, sem0), (buf1, sem1))):
                j = j2 * 2 + b
                pltpu.make_async_copy(ht.at[src_v.at[j]], buf, sem).wait()
                pltpu.sync_copy(buf, acc.at[dst_v.at[j]], add=True)
                nj = j + 2

                @pl.when(nj < CPH)
                def _():
                    pltpu.make_async_copy(ht.at[src_v.at[nj]], buf, sem).start()
            return carry
        lax.fori_loop(0, CPH // 2, body, 0)

    plsc.subcore_barrier()
    off = 0
    for sz in _RSZ:
        pltpu.sync_copy(acc.at[pl.ds(r0 + off, sz)], buf0.at[pl.ds(0, sz)])
        pltpu.sync_copy(buf0.at[pl.ds(0, sz)], out.at[c, pl.ds(r0 + off, sz)])
        off += sz


_RB = 1024


def _dinv_of(degp_blk):
    return lax.rsqrt(jnp.sum(degp_blk, axis=0) + 1.0)


def _pre_body(degp_ref, x_ref, w_ref, out_ref):
    dinv = _dinv_of(degp_ref[...])
    h = jnp.dot(x_ref[...], w_ref[...], preferred_element_type=jnp.float32,
                precision=lax.Precision.HIGHEST)
    out_ref[...] = h * dinv[:, None]


def _mid_body(degp_ref, p_ref, ht_ref, w_ref, b_ref, out_ref):
    dinv = _dinv_of(degp_ref[...])
    a = (p_ref[0] + p_ref[1] - ht_ref[...]) * dinv[:, None] + b_ref[...]
    g = jnp.maximum(a, 0.0)
    h = jnp.dot(g, w_ref[...], preferred_element_type=jnp.float32,
                precision=lax.Precision.HIGHEST)
    out_ref[...] = h * dinv[:, None]


def _post_body(degp_ref, p_ref, ht_ref, b_ref, out_ref):
    dinv = _dinv_of(degp_ref[...])
    out_ref[...] = ((p_ref[0] + p_ref[1] - ht_ref[...]) * dinv[:, None]
                    + b_ref[...])


_degp_spec = pl.BlockSpec((NT, _RB), lambda i: (0, i))
_row_spec = pl.BlockSpec((_RB, D), lambda i: (i, 0))
_p_spec = pl.BlockSpec((NC, _RB, D), lambda i: (0, i, 0))
_w_spec = pl.BlockSpec((D, D), lambda i: (0, 0))
_b_spec = pl.BlockSpec((1, D), lambda i: (0, 0))
_GRID = (NPAD // _RB,)
_row_out = jax.ShapeDtypeStruct((NPAD, D), jnp.float32)

_pre = pl.pallas_call(
    _pre_body, grid=_GRID,
    in_specs=[_degp_spec, _row_spec, _w_spec],
    out_specs=_row_spec, out_shape=_row_out)

_mid = pl.pallas_call(
    _mid_body, grid=_GRID,
    in_specs=[_degp_spec, _p_spec, _row_spec, _w_spec, _b_spec],
    out_specs=_row_spec, out_shape=_row_out)

_post = pl.pallas_call(
    _post_body, grid=_GRID,
    in_specs=[_degp_spec, _p_spec, _row_spec, _b_spec],
    out_specs=_row_spec, out_shape=_row_out)


def kernel(x, edge_index, edge_attr, W1, b1, W2, b2):
    n, d = x.shape
    src = edge_index[0]
    dst = edge_index[1]
    e = src.shape[0]

    x_p = jnp.zeros((NPAD, d), x.dtype).at[:n].set(x)
    pad = n + jnp.arange(EPAD - e, dtype=src.dtype) % (ACCN - N)
    src2 = jnp.concatenate([src, pad]).reshape(NT * CPT, CHUNK)
    dst2 = jnp.concatenate([dst, pad]).reshape(NT * CPT, CHUNK)
    b1r = b1.reshape(1, d)
    b2r = b2.reshape(1, d)

    degp = _deg_kernel(dst2)
    ht1 = _pre(degp, x_p, W1)
    p1 = _msg_kernel(ht1, src2, dst2)
    ht2 = _mid(degp, p1, ht1, W2, b1r)
    p2 = _msg_kernel(ht2, src2, dst2)
    out = _post(degp, p2, ht2, b2r)
    return out[:n]


# speedup vs baseline: 32.0180x; 1.0073x over previous
import functools

import jax
import jax.numpy as jnp
from jax import lax
from jax.experimental import pallas as pl
from jax.experimental.pallas import tpu as pltpu
from jax.experimental.pallas import tpu_sc as plsc

N = 10000
D = 128
NPAD = 10240
NC, NS = 2, 16
NT = NC * NS
CHUNK = 64
NBUF = 4
CPT = 160
NPHASE = 4
CPH = CPT // NPHASE
EPT = CPT * CHUNK
EPAD = NT * EPT
ACCN = 10112
ROWS_PT = ACCN // NS
_RSZ = (64,) * 9 + (56,)

_mesh = plsc.VectorSubcoreMesh(
    core_axis_name="c", subcore_axis_name="s", num_cores=NC, num_subcores=NS)


@functools.partial(
    pl.kernel,
    out_type=jax.ShapeDtypeStruct((NT, NPAD), jnp.float32),
    mesh=_mesh,
    scratch_types=[
        pltpu.VMEM((CPT, CHUNK), jnp.int32),
        pltpu.VMEM((NPAD,), jnp.float32),
    ],
    compiler_params=pltpu.CompilerParams(needs_layout_passes=False),
)
def _deg_kernel(dst2, out, dst_v, deg_v):
    c = lax.axis_index("c")
    s = lax.axis_index("s")
    w = c * NS + s

    def zero(i, carry):
        deg_v[pl.ds(i * 16, 16)] = jnp.zeros((16,), jnp.float32)
        return carry
    lax.fori_loop(0, NPAD // 16, zero, 0)

    pltpu.sync_copy(dst2.at[pl.ds(w * CPT, CPT)], dst_v)
    ones = jnp.ones((16,), jnp.float32)

    def body(j, carry):
        for k in range(CHUNK // 16):
            idx = dst_v[j, pl.ds(k * 16, 16)]
            plsc.addupdate_scatter(deg_v, [idx], ones)
        return carry
    lax.fori_loop(0, CPT, body, 0)

    pltpu.sync_copy(deg_v, out.at[w])


@functools.partial(
    pl.kernel,
    out_type=jax.ShapeDtypeStruct((NC, NPAD, D), jnp.float32),
    mesh=_mesh,
    scratch_types=[
        pltpu.VMEM((CPH, CHUNK), jnp.int32),
        pltpu.VMEM((CPH, CHUNK), jnp.int32),
        [pltpu.VMEM((CHUNK, D), jnp.float32)] * NBUF,
        pltpu.VMEM_SHARED((ACCN, D), jnp.float32),
        [pltpu.SemaphoreType.DMA] * NBUF,
    ],
)
def _msg_kernel(ht, src2, dst2, out, src_v, dst_v, bufs, acc, sems):
    c = lax.axis_index("c")
    s = lax.axis_index("s")
    w = c * NS + s
    r0 = s * ROWS_PT

    stage = bufs[0]
    off = 0
    for sz in _RSZ:
        pltpu.sync_copy(ht.at[pl.ds(r0 + off, sz)], stage.at[pl.ds(0, sz)])
        pltpu.sync_copy(stage.at[pl.ds(0, sz)], acc.at[pl.ds(r0 + off, sz)])
        off += sz
    plsc.subcore_barrier()

    for phase in range(NPHASE):
        base = w * CPT + phase * CPH
        pltpu.sync_copy(src2.at[pl.ds(base, CPH)], src_v)
        pltpu.sync_copy(dst2.at[pl.ds(base, CPH)], dst_v)

        for b in range(NBUF):
            pltpu.make_async_copy(ht.at[src_v.at[b]], bufs[b], sems[b]).start()

        def body(jq, carry):
            for b in range(NBUF):
                j = jq * NBUF + b
                buf, sem = bufs[b], sems[b]
                pltpu.make_async_copy(ht.at[src_v.at[j]], buf, sem).wait()
                pltpu.sync_copy(buf, acc.at[dst_v.at[j]], add=True)
                nj = j + NBUF

                @pl.when(nj < CPH)
                def _():
                    pltpu.make_async_copy(ht.at[src_v.at[nj]], buf, sem).start()
            return carry
        lax.fori_loop(0, CPH // NBUF, body, 0)

    plsc.subcore_barrier()
    off = 0
    for sz in _RSZ:
        pltpu.sync_copy(acc.at[pl.ds(r0 + off, sz)], stage.at[pl.ds(0, sz)])
        pltpu.sync_copy(stage.at[pl.ds(0, sz)], out.at[c, pl.ds(r0 + off, sz)])
        off += sz


_RB = 1024


def _dinv_of(degp_blk):
    return lax.rsqrt(jnp.sum(degp_blk, axis=0) + 1.0)


def _pre_body(degp_ref, x_ref, w_ref, out_ref):
    dinv = _dinv_of(degp_ref[...])
    h = jnp.dot(x_ref[...], w_ref[...], preferred_element_type=jnp.float32,
                precision=lax.Precision.HIGHEST)
    out_ref[...] = h * dinv[:, None]


def _mid_body(degp_ref, p_ref, ht_ref, w_ref, b_ref, out_ref):
    dinv = _dinv_of(degp_ref[...])
    a = (p_ref[0] + p_ref[1] - ht_ref[...]) * dinv[:, None] + b_ref[...]
    g = jnp.maximum(a, 0.0)
    h = jnp.dot(g, w_ref[...], preferred_element_type=jnp.float32,
                precision=lax.Precision.HIGHEST)
    out_ref[...] = h * dinv[:, None]


def _post_body(degp_ref, p_ref, ht_ref, b_ref, out_ref):
    dinv = _dinv_of(degp_ref[...])
    out_ref[...] = ((p_ref[0] + p_ref[1] - ht_ref[...]) * dinv[:, None]
                    + b_ref[...])


_degp_spec = pl.BlockSpec((NT, _RB), lambda i: (0, i))
_row_spec = pl.BlockSpec((_RB, D), lambda i: (i, 0))
_p_spec = pl.BlockSpec((NC, _RB, D), lambda i: (0, i, 0))
_w_spec = pl.BlockSpec((D, D), lambda i: (0, 0))
_b_spec = pl.BlockSpec((1, D), lambda i: (0, 0))
_GRID = (NPAD // _RB,)
_row_out = jax.ShapeDtypeStruct((NPAD, D), jnp.float32)

_pre = pl.pallas_call(
    _pre_body, grid=_GRID,
    in_specs=[_degp_spec, _row_spec, _w_spec],
    out_specs=_row_spec, out_shape=_row_out)

_mid = pl.pallas_call(
    _mid_body, grid=_GRID,
    in_specs=[_degp_spec, _p_spec, _row_spec, _w_spec, _b_spec],
    out_specs=_row_spec, out_shape=_row_out)

_post = pl.pallas_call(
    _post_body, grid=_GRID,
    in_specs=[_degp_spec, _p_spec, _row_spec, _b_spec],
    out_specs=_row_spec, out_shape=_row_out)


def kernel(x, edge_index, edge_attr, W1, b1, W2, b2):
    n, d = x.shape
    src = edge_index[0]
    dst = edge_index[1]
    e = src.shape[0]

    x_p = jnp.zeros((NPAD, d), x.dtype).at[:n].set(x)
    pad = n + jnp.arange(EPAD - e, dtype=src.dtype) % (ACCN - N)
    src2 = jnp.concatenate([src, pad]).reshape(NT * CPT, CHUNK)
    dst2 = jnp.concatenate([dst, pad]).reshape(NT * CPT, CHUNK)
    b1r = b1.reshape(1, d)
    b2r = b2.reshape(1, d)

    degp = _deg_kernel(dst2)
    ht1 = _pre(degp, x_p, W1)
    p1 = _msg_kernel(ht1, src2, dst2)
    ht2 = _mid(degp, p1, ht1, W2, b1r)
    p2 = _msg_kernel(ht2, src2, dst2)
    out = _post(degp, p2, ht2, b2r)
    return out[:n]


# speedup vs baseline: 32.6248x; 1.0190x over previous
import functools

import jax
import jax.numpy as jnp
from jax import lax
from jax.experimental import pallas as pl
from jax.experimental.pallas import tpu as pltpu
from jax.experimental.pallas import tpu_sc as plsc

N = 10000
D = 128
NPAD = 10240
NC, NS = 2, 16
NT = NC * NS
CHUNK = 64
NBUF = 4
CPT = 160
NPHASE = 4
CPH = CPT // NPHASE
EPT = CPT * CHUNK
EPAD = NT * EPT
ACCN = 10112
ROWS_PT = ACCN // NS
_RSZ = (64,) * 9 + (56,)

_mesh = plsc.VectorSubcoreMesh(
    core_axis_name="c", subcore_axis_name="s", num_cores=NC, num_subcores=NS)


@functools.partial(
    pl.kernel,
    out_type=jax.ShapeDtypeStruct((NT, NPAD), jnp.float32),
    mesh=_mesh,
    scratch_types=[
        pltpu.VMEM((CPT, CHUNK), jnp.int32),
        pltpu.VMEM((NPAD,), jnp.float32),
    ],
    compiler_params=pltpu.CompilerParams(needs_layout_passes=False),
)
def _deg_kernel(dst2, out, dst_v, deg_v):
    c = lax.axis_index("c")
    s = lax.axis_index("s")
    w = c * NS + s

    def zero(i, carry):
        deg_v[pl.ds(i * 16, 16)] = jnp.zeros((16,), jnp.float32)
        return carry
    lax.fori_loop(0, NPAD // 16, zero, 0)

    pltpu.sync_copy(dst2.at[pl.ds(w * CPT, CPT)], dst_v)
    ones = jnp.ones((16,), jnp.float32)

    def body(j, carry):
        for k in range(CHUNK // 16):
            idx = dst_v[j, pl.ds(k * 16, 16)]
            plsc.addupdate_scatter(deg_v, [idx], ones)
        return carry
    lax.fori_loop(0, CPT, body, 0)

    pltpu.sync_copy(deg_v, out.at[w])


@functools.partial(
    pl.kernel,
    out_type=jax.ShapeDtypeStruct((NC, NPAD, D), jnp.float32),
    mesh=_mesh,
    scratch_types=[
        pltpu.VMEM((CPH, CHUNK), jnp.int32),
        pltpu.VMEM((CPH, CHUNK), jnp.int32),
        [pltpu.VMEM((CHUNK, D), jnp.float32)] * NBUF,
        pltpu.VMEM_SHARED((ACCN, D), jnp.float32),
        [pltpu.SemaphoreType.DMA] * NBUF,
    ],
)
def _msg_kernel(ht, src2, dst2, out, src_v, dst_v, bufs, acc, sems):
    c = lax.axis_index("c")
    s = lax.axis_index("s")
    w = c * NS + s
    r0 = s * ROWS_PT

    stage = bufs[0]
    off = 0
    for sz in _RSZ:
        pltpu.sync_copy(ht.at[pl.ds(r0 + off, sz)], stage.at[pl.ds(0, sz)])
        pltpu.sync_copy(stage.at[pl.ds(0, sz)], acc.at[pl.ds(r0 + off, sz)])
        off += sz
    plsc.subcore_barrier()

    for phase in range(NPHASE):
        base = w * CPT + phase * CPH
        pltpu.sync_copy(src2.at[pl.ds(base, CPH)], src_v)
        pltpu.sync_copy(dst2.at[pl.ds(base, CPH)], dst_v)

        for b in range(NBUF):
            pltpu.make_async_copy(ht.at[src_v.at[b]], bufs[b], sems[b]).start()

        def body(jq, carry):
            for b in range(NBUF):
                j = jq * NBUF + b
                buf, sem = bufs[b], sems[b]
                pltpu.make_async_copy(ht.at[src_v.at[j]], buf, sem).wait()
                pltpu.sync_copy(buf, acc.at[dst_v.at[j]], add=True)
                nj = j + NBUF

                @pl.when(nj < CPH)
                def _():
                    pltpu.make_async_copy(ht.at[src_v.at[nj]], buf, sem).start()
            return carry
        lax.fori_loop(0, CPH // NBUF, body, 0)

    plsc.subcore_barrier()
    off = 0
    for sz in _RSZ:
        pltpu.sync_copy(acc.at[pl.ds(r0 + off, sz)], stage.at[pl.ds(0, sz)])
        pltpu.sync_copy(stage.at[pl.ds(0, sz)], out.at[c, pl.ds(r0 + off, sz)])
        off += sz


_RB = 1024


def _dinv_of(degp_blk):
    return lax.rsqrt(jnp.sum(degp_blk, axis=0) + 1.0)


def _pre_body(degp_ref, x_ref, w_ref, out_ref):
    dinv = _dinv_of(degp_ref[...])
    h = jnp.dot(x_ref[...], w_ref[...], preferred_element_type=jnp.float32,
                precision=lax.Precision.HIGHEST)
    out_ref[...] = h * dinv[:, None]


def _mid_body(degp_ref, p_ref, ht_ref, w_ref, b_ref, out_ref):
    dinv = _dinv_of(degp_ref[...])
    a = (p_ref[0] + p_ref[1] - ht_ref[...]) * dinv[:, None] + b_ref[...]
    g = jnp.maximum(a, 0.0)
    h = jnp.dot(g, w_ref[...], preferred_element_type=jnp.float32,
                precision=lax.Precision.HIGHEST)
    out_ref[...] = h * dinv[:, None]


def _post_body(degp_ref, p_ref, ht_ref, b_ref, out_ref):
    dinv = _dinv_of(degp_ref[...])
    out_ref[...] = ((p_ref[0] + p_ref[1] - ht_ref[...]) * dinv[:, None]
                    + b_ref[...])


_degp_spec = pl.BlockSpec((NT, _RB), lambda i: (0, i))
_row_spec = pl.BlockSpec((_RB, D), lambda i: (i, 0))
_p_spec = pl.BlockSpec((NC, _RB, D), lambda i: (0, i, 0))
_w_spec = pl.BlockSpec((D, D), lambda i: (0, 0))
_b_spec = pl.BlockSpec((1, D), lambda i: (0, 0))
_GRID = (NPAD // _RB,)
_row_out = jax.ShapeDtypeStruct((NPAD, D), jnp.float32)

_pre = pl.pallas_call(
    _pre_body, grid=_GRID,
    in_specs=[_degp_spec, _row_spec, _w_spec],
    out_specs=_row_spec, out_shape=_row_out)

_mid = pl.pallas_call(
    _mid_body, grid=_GRID,
    in_specs=[_degp_spec, _p_spec, _row_spec, _w_spec, _b_spec],
    out_specs=_row_spec, out_shape=_row_out)

_post = pl.pallas_call(
    _post_body, grid=_GRID,
    in_specs=[_degp_spec, _p_spec, _row_spec, _b_spec],
    out_specs=_row_spec, out_shape=jax.ShapeDtypeStruct((N, D), jnp.float32))


def kernel(x, edge_index, edge_attr, W1, b1, W2, b2):
    n, d = x.shape
    src = edge_index[0]
    dst = edge_index[1]
    e = src.shape[0]

    pad = n + jnp.arange(EPAD - e, dtype=src.dtype) % (ACCN - N)
    src2 = jnp.concatenate([src, pad]).reshape(NT * CPT, CHUNK)
    dst2 = jnp.concatenate([dst, pad]).reshape(NT * CPT, CHUNK)
    b1r = b1.reshape(1, d)
    b2r = b2.reshape(1, d)

    degp = _deg_kernel(dst2)
    ht1 = _pre(degp, x, W1)
    p1 = _msg_kernel(ht1, src2, dst2)
    ht2 = _mid(degp, p1, ht1, W2, b1r)
    p2 = _msg_kernel(ht2, src2, dst2)
    return _post(degp, p2, ht2, b2r)


# speedup vs baseline: 34.9760x; 1.0721x over previous
import functools

import jax
import jax.numpy as jnp
from jax import lax
from jax.experimental import pallas as pl
from jax.experimental.pallas import tpu as pltpu
from jax.experimental.pallas import tpu_sc as plsc

N = 10000
D = 128
NPAD = 10240
NC, NS = 2, 16
NT = NC * NS
CHUNK = 64
NBUF = 4
CPT = 160
NPHASE = 4
CPH = CPT // NPHASE
EPT = CPT * CHUNK
EPAD = NT * EPT
ACCN = 10112
ROWS_PT = ACCN // NS
_RSZ = (64,) * 9 + (56,)

_mesh = plsc.VectorSubcoreMesh(
    core_axis_name="c", subcore_axis_name="s", num_cores=NC, num_subcores=NS)


@functools.partial(
    pl.kernel,
    out_type=jax.ShapeDtypeStruct((NT, NPAD), jnp.float32),
    mesh=_mesh,
    scratch_types=[
        pltpu.VMEM((CPT, CHUNK), jnp.int32),
        pltpu.VMEM((NPAD,), jnp.float32),
    ],
    compiler_params=pltpu.CompilerParams(needs_layout_passes=False),
)
def _deg_kernel(dst2, out, dst_v, deg_v):
    c = lax.axis_index("c")
    s = lax.axis_index("s")
    w = c * NS + s

    def zero(i, carry):
        deg_v[pl.ds(i * 16, 16)] = jnp.zeros((16,), jnp.float32)
        return carry
    lax.fori_loop(0, NPAD // 16, zero, 0)

    pltpu.sync_copy(dst2.at[pl.ds(w * CPT, CPT)], dst_v)
    ones = jnp.ones((16,), jnp.float32)

    def body(j, carry):
        for k in range(CHUNK // 16):
            idx = dst_v[j, pl.ds(k * 16, 16)]
            plsc.addupdate_scatter(deg_v, [idx], ones)
        return carry
    lax.fori_loop(0, CPT, body, 0)

    pltpu.sync_copy(deg_v, out.at[w])


@functools.partial(
    pl.kernel,
    out_type=jax.ShapeDtypeStruct((NC, NPAD, D), jnp.float32),
    mesh=_mesh,
    scratch_types=[
        pltpu.VMEM((CPH, CHUNK), jnp.int32),
        pltpu.VMEM((CPH, CHUNK), jnp.int32),
        [pltpu.VMEM((CHUNK, D), jnp.float32)] * NBUF,
        pltpu.VMEM_SHARED((ACCN, D), jnp.float32),
        [pltpu.SemaphoreType.DMA] * NBUF,
    ],
)
def _msg_kernel(ht, src2, dst2, out, src_v, dst_v, bufs, acc, sems):
    c = lax.axis_index("c")
    s = lax.axis_index("s")
    w = c * NS + s
    r0 = s * ROWS_PT

    stage = bufs[0]
    zeros16 = jnp.zeros((16,), jnp.float32)

    def zrow(i, carry):
        for k in range(D // 16):
            stage[i, pl.ds(k * 16, 16)] = zeros16
        return carry
    lax.fori_loop(0, CHUNK, zrow, 0)
    off = 0
    for sz in _RSZ:
        pltpu.sync_copy(stage.at[pl.ds(0, sz)], acc.at[pl.ds(r0 + off, sz)])
        off += sz
    plsc.subcore_barrier()

    for phase in range(NPHASE):
        base = w * CPT + phase * CPH
        pltpu.sync_copy(src2.at[pl.ds(base, CPH)], src_v)
        pltpu.sync_copy(dst2.at[pl.ds(base, CPH)], dst_v)

        for b in range(NBUF):
            pltpu.make_async_copy(ht.at[src_v.at[b]], bufs[b], sems[b]).start()

        def body(jq, carry):
            for b in range(NBUF):
                j = jq * NBUF + b
                buf, sem = bufs[b], sems[b]
                pltpu.make_async_copy(ht.at[src_v.at[j]], buf, sem).wait()
                pltpu.sync_copy(buf, acc.at[dst_v.at[j]], add=True)
                nj = j + NBUF

                @pl.when(nj < CPH)
                def _():
                    pltpu.make_async_copy(ht.at[src_v.at[nj]], buf, sem).start()
            return carry
        lax.fori_loop(0, CPH // NBUF, body, 0)

    plsc.subcore_barrier()
    off = 0
    for sz in _RSZ:
        pltpu.sync_copy(acc.at[pl.ds(r0 + off, sz)], out.at[c, pl.ds(r0 + off, sz)])
        off += sz


_RB = 1024


def _dinv_of(degp_blk):
    return lax.rsqrt(jnp.sum(degp_blk, axis=0) + 1.0)


def _pre_body(degp_ref, x_ref, w_ref, out_ref):
    dinv = _dinv_of(degp_ref[...])
    h = jnp.dot(x_ref[...], w_ref[...], preferred_element_type=jnp.float32,
                precision=lax.Precision.HIGHEST)
    out_ref[...] = h * dinv[:, None]


def _mid_body(degp_ref, p_ref, ht_ref, w_ref, b_ref, out_ref):
    dinv = _dinv_of(degp_ref[...])
    a = (p_ref[0] + p_ref[1] + ht_ref[...]) * dinv[:, None] + b_ref[...]
    g = jnp.maximum(a, 0.0)
    h = jnp.dot(g, w_ref[...], preferred_element_type=jnp.float32,
                precision=lax.Precision.HIGHEST)
    out_ref[...] = h * dinv[:, None]


def _post_body(degp_ref, p_ref, ht_ref, b_ref, out_ref):
    dinv = _dinv_of(degp_ref[...])
    out_ref[...] = ((p_ref[0] + p_ref[1] + ht_ref[...]) * dinv[:, None]
                    + b_ref[...])


_degp_spec = pl.BlockSpec((NT, _RB), lambda i: (0, i))
_row_spec = pl.BlockSpec((_RB, D), lambda i: (i, 0))
_p_spec = pl.BlockSpec((NC, _RB, D), lambda i: (0, i, 0))
_w_spec = pl.BlockSpec((D, D), lambda i: (0, 0))
_b_spec = pl.BlockSpec((1, D), lambda i: (0, 0))
_GRID = (NPAD // _RB,)
_row_out = jax.ShapeDtypeStruct((NPAD, D), jnp.float32)

_pre = pl.pallas_call(
    _pre_body, grid=_GRID,
    in_specs=[_degp_spec, _row_spec, _w_spec],
    out_specs=_row_spec, out_shape=_row_out)

_mid = pl.pallas_call(
    _mid_body, grid=_GRID,
    in_specs=[_degp_spec, _p_spec, _row_spec, _w_spec, _b_spec],
    out_specs=_row_spec, out_shape=_row_out)

_post = pl.pallas_call(
    _post_body, grid=_GRID,
    in_specs=[_degp_spec, _p_spec, _row_spec, _b_spec],
    out_specs=_row_spec, out_shape=jax.ShapeDtypeStruct((N, D), jnp.float32))


def kernel(x, edge_index, edge_attr, W1, b1, W2, b2):
    n, d = x.shape
    src = edge_index[0]
    dst = edge_index[1]
    e = src.shape[0]

    pad = n + jnp.arange(EPAD - e, dtype=src.dtype) % (ACCN - N)
    src2 = jnp.concatenate([src, pad]).reshape(NT * CPT, CHUNK)
    dst2 = jnp.concatenate([dst, pad]).reshape(NT * CPT, CHUNK)
    b1r = b1.reshape(1, d)
    b2r = b2.reshape(1, d)

    degp = _deg_kernel(dst2)
    ht1 = _pre(degp, x, W1)
    p1 = _msg_kernel(ht1, src2, dst2)
    ht2 = _mid(degp, p1, ht1, W2, b1r)
    p2 = _msg_kernel(ht2, src2, dst2)
    return _post(degp, p2, ht2, b2r)
